# baseline (speedup 1.0000x reference)
import math

import jax
import jax.numpy as jnp
from jax import lax
from jax.experimental import pallas as pl
from jax.experimental.pallas import tpu as pltpu
from jax.experimental.pallas import tpu_sc as plsc

VOCAB = 1000000
DMODEL = 64
SCALE = math.sqrt(DMODEL)

_INFO = plsc.get_sparse_core_info()
NC = _INFO.num_cores
NS = _INFO.num_subcores
NW = NC * NS
LANES = 16

CHUNK = 128


def _make_kernel(batch):
    assert batch % (NW * CHUNK) == 0
    b_per_w = batch // NW
    n_chunks = b_per_w // CHUNK
    mesh = plsc.VectorSubcoreMesh(core_axis_name="c", subcore_axis_name="s")

    def body(x_hbm, table_hbm, out_hbm, idx_v, rows_v, sem):
        wid = lax.axis_index("s") * NC + lax.axis_index("c")
        base = wid * b_per_w

        def chunk_body(g, carry):
            off = base + g * CHUNK
            pltpu.sync_copy(x_hbm.at[pl.ds(off, CHUNK)], idx_v)
            pltpu.async_copy(table_hbm.at[idx_v], rows_v, sem).wait()

            def row_body(r, c):
                for j in range(DMODEL // LANES):
                    sl = pl.ds(j * LANES, LANES)
                    rows_v[r, sl] = rows_v[r, sl] * SCALE
                return c

            lax.fori_loop(0, CHUNK, row_body, 0)
            pltpu.sync_copy(rows_v, out_hbm.at[pl.ds(off, CHUNK)])
            return carry

        lax.fori_loop(0, n_chunks, chunk_body, 0)

    return pl.kernel(
        body,
        out_type=jax.ShapeDtypeStruct((batch, DMODEL), jnp.float32),
        mesh=mesh,
        scratch_types=[
            pltpu.VMEM((CHUNK,), jnp.int32),
            pltpu.VMEM((CHUNK, DMODEL), jnp.float32),
            pltpu.SemaphoreType.DMA,
        ],
        compiler_params=pltpu.CompilerParams(use_tc_tiling_on_sc=False),
    )


def kernel(x, table):
    orig_shape = x.shape
    flat = x.reshape(-1).astype(jnp.int32)
    out = _make_kernel(flat.shape[0])(flat, table)
    return out.reshape(*orig_shape, DMODEL)


# speedup vs baseline: 1.2534x; 1.2534x over previous
import math

import jax
import jax.numpy as jnp
from jax import lax
from jax.experimental import pallas as pl
from jax.experimental.pallas import tpu as pltpu
from jax.experimental.pallas import tpu_sc as plsc

VOCAB = 1000000
DMODEL = 64
SCALE = math.sqrt(DMODEL)

_INFO = plsc.get_sparse_core_info()
NC = _INFO.num_cores
NS = _INFO.num_subcores
NW = NC * NS
LANES = 16

CHUNK = 128
NBUF = 4


def _scale_chunk(rows_b):
    @plsc.parallel_loop(0, CHUNK, step=1, unroll=8)
    def _(r):
        for j in range(DMODEL // LANES):
            sl = pl.ds(j * LANES, LANES)
            rows_b[r, sl] = rows_b[r, sl] * SCALE


def _make_kernel(batch):
    assert batch % (NW * CHUNK) == 0
    b_per_w = batch // NW
    n_chunks = b_per_w // CHUNK
    assert n_chunks % NBUF == 0 and n_chunks // NBUF >= 2
    n_outer = n_chunks // NBUF
    mesh = plsc.VectorSubcoreMesh(core_axis_name="c", subcore_axis_name="s")

    def body(x_hbm, table_hbm, out_hbm, idx_v, rows_v, idx_sems, gat_sems,
             st_sems):
        wid = lax.axis_index("s") * NC + lax.axis_index("c")
        base = wid * b_per_w

        def idx_start(c, slot):
            pltpu.async_copy(x_hbm.at[pl.ds(base + c * CHUNK, CHUNK)],
                             idx_v.at[slot], idx_sems.at[slot])

        def idx_wait(c, slot):
            pltpu.make_async_copy(x_hbm.at[pl.ds(base + c * CHUNK, CHUNK)],
                                  idx_v.at[slot], idx_sems.at[slot]).wait()

        def gat_start(slot):
            pltpu.async_copy(table_hbm.at[idx_v.at[slot]], rows_v.at[slot],
                             gat_sems.at[slot])

        def gat_wait(slot):
            pltpu.make_async_copy(table_hbm.at[idx_v.at[slot]],
                                  rows_v.at[slot], gat_sems.at[slot]).wait()

        def st_start(c, slot):
            pltpu.async_copy(rows_v.at[slot],
                             out_hbm.at[pl.ds(base + c * CHUNK, CHUNK)],
                             st_sems.at[slot])

        def st_wait(c, slot):
            pltpu.make_async_copy(rows_v.at[slot],
                                  out_hbm.at[pl.ds(base + c * CHUNK, CHUNK)],
                                  st_sems.at[slot]).wait()

        for c in range(NBUF - 1):
            idx_start(c, c)
        idx_wait(0, 0)
        gat_start(0)
        idx_start(NBUF - 1, NBUF - 1)
        for g in range(1, NBUF):
            idx_wait(g, g)
            gat_start(g)
            gat_wait(g - 1)
            idx_start(g + NBUF - 1, g - 1)
            _scale_chunk(rows_v.at[g - 1])
            st_start(g - 1, g - 1)

        def outer(g0, carry):
            for b in range(NBUF):
                g = g0 * NBUF + b
                st_wait(g - NBUF, b)
                idx_wait(g, b)
                gat_start(b)
                pb = (b - 1) % NBUF
                gat_wait(pb)
                idx_start(jnp.minimum(g + NBUF - 1, n_chunks - 1), pb)
                _scale_chunk(rows_v.at[pb])
                st_start(g - 1, pb)
            return carry

        lax.fori_loop(1, n_outer, outer, 0)

        last = NBUF - 1
        gat_wait(last)
        _scale_chunk(rows_v.at[last])
        st_start(n_chunks - 1, last)
        for b in range(NBUF):
            st_wait(n_chunks - NBUF + b, b)
        for s in range(NBUF - 1):
            idx_wait(0, s)

    return pl.kernel(
        body,
        out_type=jax.ShapeDtypeStruct((batch, DMODEL), jnp.float32),
        mesh=mesh,
        scratch_types=[
            pltpu.VMEM((NBUF, CHUNK), jnp.int32),
            pltpu.VMEM((NBUF, CHUNK, DMODEL), jnp.float32),
            pltpu.SemaphoreType.DMA((NBUF,)),
            pltpu.SemaphoreType.DMA((NBUF,)),
            pltpu.SemaphoreType.DMA((NBUF,)),
        ],
        compiler_params=pltpu.CompilerParams(use_tc_tiling_on_sc=False),
    )


def kernel(x, table):
    orig_shape = x.shape
    flat = x.reshape(-1).astype(jnp.int32)
    out = _make_kernel(flat.shape[0])(flat, table)
    return out.reshape(*orig_shape, DMODEL)


# speedup vs baseline: 1.2557x; 1.0018x over previous
import math

import jax
import jax.numpy as jnp
from jax import lax
from jax.experimental import pallas as pl
from jax.experimental.pallas import tpu as pltpu
from jax.experimental.pallas import tpu_sc as plsc

VOCAB = 1000000
DMODEL = 64
SCALE = math.sqrt(DMODEL)

_INFO = plsc.get_sparse_core_info()
NC = _INFO.num_cores
NS = _INFO.num_subcores
NW = NC * NS
LANES = 16

CHUNK = 128
NBUF = 4
DH = DMODEL // 8
NJ = 200


def _transpose_scale(rows_b, trans_b):
    riota = lax.iota(jnp.int32, 16)

    @plsc.parallel_loop(0, DMODEL, step=1, unroll=2)
    def _(d):
        dh = d // 8
        dl = d % 8
        col = jnp.full((16,), d, jnp.int32)
        for ilg in range(CHUNK // LANES):
            vals = plsc.load_gather(rows_b, [riota + ilg * LANES, col])
            trans_b[dh, dl, pl.ds(ilg * LANES, LANES)] = vals * SCALE


def _make_kernel():
    mesh = plsc.VectorSubcoreMesh(core_axis_name="c", subcore_axis_name="s")

    def body(xt_hbm, table_hbm, out_hbm, idx_all, rows_v, trans_v, gat_sems,
             st_sems):
        wid = lax.axis_index("s") * NC + lax.axis_index("c")

        pltpu.sync_copy(xt_hbm.at[:, wid], idx_all)

        def gat_start(j, slot):
            pltpu.async_copy(table_hbm.at[idx_all.at[j]], rows_v.at[slot],
                             gat_sems.at[slot])

        def gat_wait(j, slot):
            pltpu.make_async_copy(table_hbm.at[idx_all.at[j]],
                                  rows_v.at[slot], gat_sems.at[slot]).wait()

        def st_start(j, slot):
            pltpu.async_copy(trans_v.at[slot], out_hbm.at[j, :, wid],
                             st_sems.at[slot])

        def st_wait(j, slot):
            pltpu.make_async_copy(trans_v.at[slot], out_hbm.at[j, :, wid],
                                  st_sems.at[slot]).wait()

        def ts(slot):
            _transpose_scale(rows_v.at[slot], trans_v.at[slot])

        gat_start(0, 0)
        for j in range(1, NBUF):
            gat_start(j, j)
            gat_wait(j - 1, j - 1)
            ts(j - 1)
            st_start(j - 1, j - 1)

        def outer(jq, carry):
            for b in range(NBUF):
                j = jq * NBUF + b
                st_wait(j - NBUF, b)
                gat_start(j, b)
                pb = (b - 1) % NBUF
                gat_wait(j - 1, pb)
                ts(pb)
                st_start(j - 1, pb)
            return carry

        lax.fori_loop(1, NJ // NBUF, outer, 0)

        last = NBUF - 1
        gat_wait(NJ - 1, last)
        ts(last)
        st_start(NJ - 1, last)
        for b in range(NBUF):
            st_wait(NJ - NBUF + b, b)

    return pl.kernel(
        body,
        out_type=jax.ShapeDtypeStruct((NJ, DH, NW, 8, CHUNK), jnp.float32),
        mesh=mesh,
        scratch_types=[
            pltpu.VMEM((NJ, CHUNK), jnp.int32),
            pltpu.VMEM((NBUF, CHUNK, DMODEL), jnp.float32),
            pltpu.VMEM((NBUF, DH, 8, CHUNK), jnp.float32),
            pltpu.SemaphoreType.DMA((NBUF,)),
            pltpu.SemaphoreType.DMA((NBUF,)),
        ],
        compiler_params=pltpu.CompilerParams(use_tc_tiling_on_sc=False,
                                             needs_layout_passes=False),
    )


def kernel(x, table):
    n_i, n_j = x.shape
    assert n_i == NW * CHUNK and n_j == NJ
    xt = x.astype(jnp.int32).transpose((1, 0)).reshape(NJ, NW, CHUNK)
    out5 = _make_kernel()(xt, table)
    return out5.transpose((2, 4, 0, 1, 3)).reshape(n_i, n_j, DMODEL)
